# Initial kernel scaffold; baseline (speedup 1.0000x reference)
import jax
import jax.numpy as jnp
from jax.experimental import pallas as pl


def kernel(x, edge_index, W1, b1, W2, b2):
    raise NotImplementedError("write your pallas kernel here")


# speedup vs baseline: 9.1981x; 9.1981x over previous
import functools

import jax
import jax.numpy as jnp
from jax import lax
from jax.experimental import pallas as pl
from jax.experimental.pallas import tpu as pltpu
from jax.experimental.pallas import tpu_sc as plsc

NC = 2
NS = 16
NW = NC * NS
LANES = 16
CHUNK = 128
NBUF = 4
ROWBLK = 1280


def _mesh():
    return plsc.VectorSubcoreMesh(core_axis_name="c", subcore_axis_name="s")


@functools.lru_cache(maxsize=None)
def _sc_degree(n_pad: int, nch: int):
    rows_per_tile = n_pad // NS
    zcopies = rows_per_tile // CHUNK

    scratch = [
        pltpu.VMEM((nch, CHUNK), jnp.int32),
        pltpu.VMEM((CHUNK, LANES), jnp.float32),
        pltpu.VMEM((CHUNK, LANES), jnp.float32),
        pltpu.VMEM_SHARED((n_pad, LANES), jnp.float32),
        pltpu.SemaphoreType.DMA,
    ]

    @functools.partial(
        pl.kernel,
        out_type=jax.ShapeDtypeStruct((NC, n_pad, LANES), jnp.float32),
        mesh=_mesh(),
        scratch_types=scratch,
        compiler_params=pltpu.CompilerParams(use_tc_tiling_on_sc=False),
    )
    def deg_k(dst_hbm, out_hbm, dst_v, ones_v, zero_v, acc, sem):
        c = lax.axis_index("c")
        s = lax.axis_index("s")
        wid = s * NC + c
        pltpu.sync_copy(dst_hbm.at[wid], dst_v)

        ones = jnp.ones((LANES,), jnp.float32)
        zeros = jnp.zeros((LANES,), jnp.float32)

        def fbody(i, carry):
            ones_v[i, :] = ones
            zero_v[i, :] = zeros
            return carry

        lax.fori_loop(0, CHUNK, fbody, 0)
        for k in range(zcopies):
            pltpu.sync_copy(
                zero_v, acc.at[pl.ds(s * rows_per_tile + k * CHUNK, CHUNK)])
        plsc.subcore_barrier()

        def fire(j, carry):
            pltpu.async_copy(ones_v, acc.at[dst_v.at[j]], sem, add=True)
            return carry

        lax.fori_loop(0, nch, fire, 0)

        def drain(j, carry):
            pltpu.make_async_copy(ones_v, acc.at[dst_v.at[j]], sem).wait()
            return carry

        lax.fori_loop(0, nch, drain, 0)
        plsc.subcore_barrier()
        pltpu.sync_copy(
            acc.at[pl.ds(s * rows_per_tile, rows_per_tile)],
            out_hbm.at[c].at[pl.ds(s * rows_per_tile, rows_per_tile)])

    return deg_k


@functools.lru_cache(maxsize=None)
def _sc_agg(n_pad: int, d: int, nch: int):
    dh = d // 2
    rows_per_tile = n_pad // NS
    zcopies = rows_per_tile // CHUNK

    scratch = [
        pltpu.VMEM((nch, CHUNK), jnp.int32),
        pltpu.VMEM((nch, CHUNK), jnp.int32),
        pltpu.VMEM((NBUF, CHUNK, dh), jnp.float32),
        pltpu.VMEM((CHUNK, dh), jnp.float32),
        pltpu.VMEM_SHARED((n_pad, dh), jnp.float32),
    ] + [pltpu.SemaphoreType.DMA] * NBUF

    @functools.partial(
        pl.kernel,
        out_type=jax.ShapeDtypeStruct((NC, 2, n_pad, dh), jnp.float32),
        mesh=_mesh(),
        scratch_types=scratch,
        compiler_params=pltpu.CompilerParams(use_tc_tiling_on_sc=False),
    )
    def agg_k(xw_hbm, src_hbm, dst_hbm, out_hbm, src_v, dst_v, buf, zbuf, acc,
              *gsems):
        c = lax.axis_index("c")
        s = lax.axis_index("s")
        wid = s * NC + c
        pltpu.sync_copy(src_hbm.at[wid], src_v)
        pltpu.sync_copy(dst_hbm.at[wid], dst_v)

        zeros = jnp.zeros((LANES,), jnp.float32)

        def zbody(i, carry):
            for k in range(dh // LANES):
                zbuf[i, pl.ds(k * LANES, LANES)] = zeros
            return carry

        lax.fori_loop(0, CHUNK, zbody, 0)

        for half in range(2):
            xw_h = xw_hbm.at[half]
            for k in range(zcopies):
                pltpu.sync_copy(
                    zbuf, acc.at[pl.ds(s * rows_per_tile + k * CHUNK, CHUNK)])
            plsc.subcore_barrier()

            for b in range(NBUF):
                pltpu.async_copy(xw_h.at[src_v.at[b]], buf.at[b], gsems[b])

            def step(jo, carry):
                for b in range(NBUF):
                    j = jo * NBUF + b
                    pltpu.make_async_copy(
                        xw_h.at[src_v.at[j]], buf.at[b], gsems[b]).wait()
                    pltpu.sync_copy(buf.at[b], acc.at[dst_v.at[j]], add=True)
                    jn = j + NBUF

                    @pl.when(jn < nch)
                    def _():
                        pltpu.async_copy(
                            xw_h.at[src_v.at[jn]], buf.at[b], gsems[b])

                return carry

            lax.fori_loop(0, nch // NBUF, step, 0)
            plsc.subcore_barrier()
            pltpu.sync_copy(
                acc.at[pl.ds(s * rows_per_tile, rows_per_tile)],
                out_hbm.at[c].at[half].at[pl.ds(s * rows_per_tile,
                                                rows_per_tile)])
            plsc.subcore_barrier()

    return agg_k


def _dis_block(dp_ref):
    deg = dp_ref[0, :, 0:1] + dp_ref[1, :, 0:1] + 1.0
    return lax.rsqrt(deg)


def _deg_spec():
    return pl.BlockSpec((NC, ROWBLK, LANES), lambda i: (0, i, 0))


def _tc_scale_matmul(x_pad, W, deg_parts):
    n_pad, d = x_pad.shape
    dh = d // 2

    def body(x_ref, w_ref, dp_ref, o_ref):
        dis = _dis_block(dp_ref)
        xw = jnp.dot(x_ref[...], w_ref[0],
                     precision=lax.Precision.HIGHEST,
                     preferred_element_type=jnp.float32)
        o_ref[0] = xw * dis

    return pl.pallas_call(
        body,
        grid=(2, n_pad // ROWBLK),
        in_specs=[
            pl.BlockSpec((ROWBLK, d), lambda j, i: (i, 0)),
            pl.BlockSpec((1, d, dh), lambda j, i: (j, 0, 0)),
            pl.BlockSpec((NC, ROWBLK, LANES), lambda j, i: (0, i, 0)),
        ],
        out_specs=pl.BlockSpec((1, ROWBLK, dh), lambda j, i: (j, i, 0)),
        out_shape=jax.ShapeDtypeStruct((2, n_pad, dh), jnp.float32),
    )(x_pad, W, deg_parts)


def _agg_block(p_ref, xw_ref, dis):
    return jnp.concatenate(
        [(p_ref[0, h] + p_ref[1, h] + xw_ref[h]) * dis for h in range(2)],
        axis=1)


def _tc_mid(parts, xw_s, deg_parts, b, W):
    _, n_pad, dh = xw_s.shape
    d = 2 * dh

    def body(p_ref, xw_ref, dp_ref, b_ref, w_ref, o_ref):
        dis = _dis_block(dp_ref)
        h = jnp.maximum(_agg_block(p_ref, xw_ref, dis) + b_ref[...], 0.0)
        o_ref[0] = jnp.dot(h, w_ref[0],
                           precision=lax.Precision.HIGHEST,
                           preferred_element_type=jnp.float32) * dis

    return pl.pallas_call(
        body,
        grid=(2, n_pad // ROWBLK),
        in_specs=[
            pl.BlockSpec((NC, 2, ROWBLK, dh), lambda j, i: (0, 0, i, 0)),
            pl.BlockSpec((2, ROWBLK, dh), lambda j, i: (0, i, 0)),
            pl.BlockSpec((NC, ROWBLK, LANES), lambda j, i: (0, i, 0)),
            pl.BlockSpec((1, d), lambda j, i: (0, 0)),
            pl.BlockSpec((1, d, dh), lambda j, i: (j, 0, 0)),
        ],
        out_specs=pl.BlockSpec((1, ROWBLK, dh), lambda j, i: (j, i, 0)),
        out_shape=jax.ShapeDtypeStruct((2, n_pad, dh), jnp.float32),
    )(parts, xw_s, deg_parts, b, W)


def _tc_final(parts, xw_s, deg_parts, b):
    _, n_pad, dh = xw_s.shape
    d = 2 * dh

    def body(p_ref, xw_ref, dp_ref, b_ref, o_ref):
        dis = _dis_block(dp_ref)
        o_ref[...] = _agg_block(p_ref, xw_ref, dis) + b_ref[...]

    return pl.pallas_call(
        body,
        grid=(n_pad // ROWBLK,),
        in_specs=[
            pl.BlockSpec((NC, 2, ROWBLK, dh), lambda i: (0, 0, i, 0)),
            pl.BlockSpec((2, ROWBLK, dh), lambda i: (0, i, 0)),
            _deg_spec(),
            pl.BlockSpec((1, d), lambda i: (0, 0)),
        ],
        out_specs=pl.BlockSpec((ROWBLK, d), lambda i: (i, 0)),
        out_shape=jax.ShapeDtypeStruct((n_pad, d), jnp.float32),
    )(parts, xw_s, deg_parts, b)


def kernel(x, edge_index, W1, b1, W2, b2):
    n, d = x.shape
    e = edge_index.shape[1]

    n_pad = -(-(n + 1) // ROWBLK) * ROWBLK
    epw = -(-e // NW)
    nch = -(-epw // CHUNK)
    nch = -(-nch // NBUF) * NBUF
    e_pad = NW * nch * CHUNK

    pad = jnp.full((e_pad - e,), n, dtype=edge_index.dtype)
    src = jnp.concatenate([edge_index[0], pad]).reshape(NW, nch, CHUNK)
    dst = jnp.concatenate([edge_index[1], pad]).reshape(NW, nch, CHUNK)

    x_pad = jnp.zeros((n_pad, d), jnp.float32).at[:n].set(x)

    deg_parts = _sc_degree(n_pad, nch)(dst)

    b1r = b1.reshape(1, d)
    b2r = b2.reshape(1, d)
    dh = d // 2
    W1h = jnp.stack([W1[:, :dh], W1[:, dh:]])
    W2h = jnp.stack([W2[:, :dh], W2[:, dh:]])

    xw1s = _tc_scale_matmul(x_pad, W1h, deg_parts)
    parts1 = _sc_agg(n_pad, d, nch)(xw1s, src, dst)
    xw2s = _tc_mid(parts1, xw1s, deg_parts, b1r, W2h)
    parts2 = _sc_agg(n_pad, d, nch)(xw2s, src, dst)
    out = _tc_final(parts2, xw2s, deg_parts, b2r)
    return out[:n]


# speedup vs baseline: 9.2097x; 1.0013x over previous
import functools

import jax
import jax.numpy as jnp
from jax import lax
from jax.experimental import pallas as pl
from jax.experimental.pallas import tpu as pltpu
from jax.experimental.pallas import tpu_sc as plsc

NC = 2
NS = 16
NW = NC * NS
LANES = 16
CHUNK = 128
NBUF = 2
ROWBLK = 1280


def _mesh():
    return plsc.VectorSubcoreMesh(core_axis_name="c", subcore_axis_name="s")


@functools.lru_cache(maxsize=None)
def _sc_degree(n_pad: int, nch: int):
    rows_per_tile = n_pad // NS
    zcopies = rows_per_tile // CHUNK

    scratch = [
        pltpu.VMEM((nch, CHUNK), jnp.int32),
        pltpu.VMEM((CHUNK, LANES), jnp.float32),
        pltpu.VMEM((CHUNK, LANES), jnp.float32),
        pltpu.VMEM_SHARED((n_pad, LANES), jnp.float32),
        pltpu.SemaphoreType.DMA,
    ]

    @functools.partial(
        pl.kernel,
        out_type=jax.ShapeDtypeStruct((NC, n_pad, LANES), jnp.float32),
        mesh=_mesh(),
        scratch_types=scratch,
        compiler_params=pltpu.CompilerParams(use_tc_tiling_on_sc=False),
    )
    def deg_k(dst_hbm, out_hbm, dst_v, ones_v, zero_v, acc, sem):
        c = lax.axis_index("c")
        s = lax.axis_index("s")
        wid = s * NC + c
        pltpu.sync_copy(dst_hbm.at[wid], dst_v)

        ones = jnp.ones((LANES,), jnp.float32)
        zeros = jnp.zeros((LANES,), jnp.float32)

        def fbody(i, carry):
            ones_v[i, :] = ones
            zero_v[i, :] = zeros
            return carry

        lax.fori_loop(0, CHUNK, fbody, 0)
        for k in range(zcopies):
            pltpu.sync_copy(
                zero_v, acc.at[pl.ds(s * rows_per_tile + k * CHUNK, CHUNK)])
        plsc.subcore_barrier()

        def fire(j, carry):
            pltpu.async_copy(ones_v, acc.at[dst_v.at[j]], sem, add=True)
            return carry

        lax.fori_loop(0, nch, fire, 0)

        def drain(j, carry):
            pltpu.make_async_copy(ones_v, acc.at[dst_v.at[j]], sem).wait()
            return carry

        lax.fori_loop(0, nch, drain, 0)
        plsc.subcore_barrier()
        pltpu.sync_copy(
            acc.at[pl.ds(s * rows_per_tile, rows_per_tile)],
            out_hbm.at[c].at[pl.ds(s * rows_per_tile, rows_per_tile)])

    return deg_k


@functools.lru_cache(maxsize=None)
def _sc_agg(n_pad: int, d: int, nch: int):
    dh = d // 2
    rows_per_tile = n_pad // NS
    zcopies = rows_per_tile // CHUNK
    nb2 = 2 * NBUF

    scratch = [
        pltpu.VMEM((nch, CHUNK), jnp.int32),
        pltpu.VMEM((nch, CHUNK), jnp.int32),
        pltpu.VMEM((nb2, CHUNK, dh), jnp.float32),
        pltpu.VMEM((CHUNK, dh), jnp.float32),
        pltpu.VMEM_SHARED((n_pad, dh), jnp.float32),
    ] + [pltpu.SemaphoreType.DMA] * (2 * nb2)

    @functools.partial(
        pl.kernel,
        out_type=jax.ShapeDtypeStruct((NC, 2, n_pad, dh), jnp.float32),
        mesh=_mesh(),
        scratch_types=scratch,
        compiler_params=pltpu.CompilerParams(use_tc_tiling_on_sc=False),
    )
    def agg_k(xw_hbm, src_hbm, dst_hbm, out_hbm, src_v, dst_v, buf, zbuf, acc,
              *sems):
        gsems = sems[:nb2]
        ssems = sems[nb2:]
        c = lax.axis_index("c")
        s = lax.axis_index("s")
        wid = s * NC + c
        pltpu.sync_copy(src_hbm.at[wid], src_v)
        pltpu.sync_copy(dst_hbm.at[wid], dst_v)

        zeros = jnp.zeros((LANES,), jnp.float32)

        def zbody(i, carry):
            for k in range(dh // LANES):
                zbuf[i, pl.ds(k * LANES, LANES)] = zeros
            return carry

        lax.fori_loop(0, CHUNK, zbody, 0)

        for half in range(2):
            xw_h = xw_hbm.at[half]
            for k in range(zcopies):
                pltpu.sync_copy(
                    zbuf, acc.at[pl.ds(s * rows_per_tile + k * CHUNK, CHUNK)])
            plsc.subcore_barrier()

            for b in range(NBUF):
                pltpu.async_copy(xw_h.at[src_v.at[b]], buf.at[b], gsems[b])

            def step(jo, carry):
                for u in range(nb2):
                    j = jo * nb2 + u
                    b = u
                    pltpu.make_async_copy(
                        xw_h.at[src_v.at[j]], buf.at[b], gsems[b]).wait()
                    pltpu.async_copy(
                        buf.at[b], acc.at[dst_v.at[j]], ssems[b], add=True)
                    jn = j + NBUF
                    bn = (u + NBUF) % nb2

                    @pl.when(jn < nch)
                    def _():
                        @pl.when(jn >= nb2)
                        def _():
                            pltpu.make_async_copy(
                                buf.at[bn], acc.at[dst_v.at[jn]],
                                ssems[bn]).wait()

                        pltpu.async_copy(
                            xw_h.at[src_v.at[jn]], buf.at[bn], gsems[bn])

                return carry

            lax.fori_loop(0, nch // nb2, step, 0)

            for u in range(nb2):
                j = nch - nb2 + u
                pltpu.make_async_copy(
                    buf.at[u], acc.at[dst_v.at[j]], ssems[u]).wait()

            plsc.subcore_barrier()
            pltpu.sync_copy(
                acc.at[pl.ds(s * rows_per_tile, rows_per_tile)],
                out_hbm.at[c].at[half].at[pl.ds(s * rows_per_tile,
                                                rows_per_tile)])
            plsc.subcore_barrier()

    return agg_k


def _dis_block(dp_ref):
    deg = dp_ref[0, :, 0:1] + dp_ref[1, :, 0:1] + 1.0
    return lax.rsqrt(deg)


def _deg_spec():
    return pl.BlockSpec((NC, ROWBLK, LANES), lambda i: (0, i, 0))


def _tc_scale_matmul(x_pad, W, deg_parts):
    n_pad, d = x_pad.shape
    dh = d // 2

    def body(x_ref, w_ref, dp_ref, o_ref):
        dis = _dis_block(dp_ref)
        xw = jnp.dot(x_ref[...], w_ref[0],
                     precision=lax.Precision.HIGHEST,
                     preferred_element_type=jnp.float32)
        o_ref[0] = xw * dis

    return pl.pallas_call(
        body,
        grid=(2, n_pad // ROWBLK),
        in_specs=[
            pl.BlockSpec((ROWBLK, d), lambda j, i: (i, 0)),
            pl.BlockSpec((1, d, dh), lambda j, i: (j, 0, 0)),
            pl.BlockSpec((NC, ROWBLK, LANES), lambda j, i: (0, i, 0)),
        ],
        out_specs=pl.BlockSpec((1, ROWBLK, dh), lambda j, i: (j, i, 0)),
        out_shape=jax.ShapeDtypeStruct((2, n_pad, dh), jnp.float32),
    )(x_pad, W, deg_parts)


def _agg_block(p_ref, xw_ref, dis):
    return jnp.concatenate(
        [(p_ref[0, h] + p_ref[1, h] + xw_ref[h]) * dis for h in range(2)],
        axis=1)


def _tc_mid(parts, xw_s, deg_parts, b, W):
    _, n_pad, dh = xw_s.shape
    d = 2 * dh

    def body(p_ref, xw_ref, dp_ref, b_ref, w_ref, o_ref):
        dis = _dis_block(dp_ref)
        h = jnp.maximum(_agg_block(p_ref, xw_ref, dis) + b_ref[...], 0.0)
        o_ref[0] = jnp.dot(h, w_ref[0],
                           precision=lax.Precision.HIGHEST,
                           preferred_element_type=jnp.float32) * dis

    return pl.pallas_call(
        body,
        grid=(2, n_pad // ROWBLK),
        in_specs=[
            pl.BlockSpec((NC, 2, ROWBLK, dh), lambda j, i: (0, 0, i, 0)),
            pl.BlockSpec((2, ROWBLK, dh), lambda j, i: (0, i, 0)),
            pl.BlockSpec((NC, ROWBLK, LANES), lambda j, i: (0, i, 0)),
            pl.BlockSpec((1, d), lambda j, i: (0, 0)),
            pl.BlockSpec((1, d, dh), lambda j, i: (j, 0, 0)),
        ],
        out_specs=pl.BlockSpec((1, ROWBLK, dh), lambda j, i: (j, i, 0)),
        out_shape=jax.ShapeDtypeStruct((2, n_pad, dh), jnp.float32),
    )(parts, xw_s, deg_parts, b, W)


def _tc_final(parts, xw_s, deg_parts, b):
    _, n_pad, dh = xw_s.shape
    d = 2 * dh

    def body(p_ref, xw_ref, dp_ref, b_ref, o_ref):
        dis = _dis_block(dp_ref)
        o_ref[...] = _agg_block(p_ref, xw_ref, dis) + b_ref[...]

    return pl.pallas_call(
        body,
        grid=(n_pad // ROWBLK,),
        in_specs=[
            pl.BlockSpec((NC, 2, ROWBLK, dh), lambda i: (0, 0, i, 0)),
            pl.BlockSpec((2, ROWBLK, dh), lambda i: (0, i, 0)),
            _deg_spec(),
            pl.BlockSpec((1, d), lambda i: (0, 0)),
        ],
        out_specs=pl.BlockSpec((ROWBLK, d), lambda i: (i, 0)),
        out_shape=jax.ShapeDtypeStruct((n_pad, d), jnp.float32),
    )(parts, xw_s, deg_parts, b)


def kernel(x, edge_index, W1, b1, W2, b2):
    n, d = x.shape
    e = edge_index.shape[1]

    n_pad = -(-(n + 1) // ROWBLK) * ROWBLK
    epw = -(-e // NW)
    nch = -(-epw // CHUNK)
    nch = -(-nch // (2 * NBUF)) * 2 * NBUF
    e_pad = NW * nch * CHUNK

    pad = jnp.full((e_pad - e,), n, dtype=edge_index.dtype)
    src = jnp.concatenate([edge_index[0], pad]).reshape(NW, nch, CHUNK)
    dst = jnp.concatenate([edge_index[1], pad]).reshape(NW, nch, CHUNK)

    x_pad = jnp.zeros((n_pad, d), jnp.float32).at[:n].set(x)

    deg_parts = _sc_degree(n_pad, nch)(dst)

    b1r = b1.reshape(1, d)
    b2r = b2.reshape(1, d)
    dh = d // 2
    W1h = jnp.stack([W1[:, :dh], W1[:, dh:]])
    W2h = jnp.stack([W2[:, :dh], W2[:, dh:]])

    xw1s = _tc_scale_matmul(x_pad, W1h, deg_parts)
    parts1 = _sc_agg(n_pad, d, nch)(xw1s, src, dst)
    xw2s = _tc_mid(parts1, xw1s, deg_parts, b1r, W2h)
    parts2 = _sc_agg(n_pad, d, nch)(xw2s, src, dst)
    out = _tc_final(parts2, xw2s, deg_parts, b2r)
    return out[:n]


# speedup vs baseline: 23.5919x; 2.5616x over previous
import functools

import jax
import jax.numpy as jnp
from jax import lax
from jax.experimental import pallas as pl
from jax.experimental.pallas import tpu as pltpu
from jax.experimental.pallas import tpu_sc as plsc

NC = 2
NS = 16
NW = NC * NS
LANES = 16
CHUNK = 128
NBUF = 2
ROWBLK = 1280


def _mesh():
    return plsc.VectorSubcoreMesh(core_axis_name="c", subcore_axis_name="s")


@functools.lru_cache(maxsize=None)
def _sc_degree(n_pad: int, nch: int):
    rows_per_tile = n_pad // NS
    zcopies = rows_per_tile // CHUNK

    scratch = [
        pltpu.VMEM((nch, CHUNK), jnp.int32),
        pltpu.VMEM((CHUNK, LANES), jnp.float32),
        pltpu.VMEM((CHUNK, LANES), jnp.float32),
        pltpu.VMEM_SHARED((n_pad, LANES), jnp.float32),
        pltpu.SemaphoreType.DMA,
    ]

    @functools.partial(
        pl.kernel,
        out_type=jax.ShapeDtypeStruct((NC, n_pad, LANES), jnp.float32),
        mesh=_mesh(),
        scratch_types=scratch,
        compiler_params=pltpu.CompilerParams(use_tc_tiling_on_sc=False),
    )
    def deg_k(dst_hbm, out_hbm, dst_v, ones_v, zero_v, acc, sem):
        c = lax.axis_index("c")
        s = lax.axis_index("s")
        wid = s * NC + c
        pltpu.sync_copy(dst_hbm.at[wid], dst_v)

        ones = jnp.ones((LANES,), jnp.float32)
        zeros = jnp.zeros((LANES,), jnp.float32)

        def fbody(i, carry):
            ones_v[i, :] = ones
            zero_v[i, :] = zeros
            return carry

        lax.fori_loop(0, CHUNK, fbody, 0)
        for k in range(zcopies):
            pltpu.sync_copy(
                zero_v, acc.at[pl.ds(s * rows_per_tile + k * CHUNK, CHUNK)])
        plsc.subcore_barrier()

        def fire(j, carry):
            pltpu.async_copy(ones_v, acc.at[dst_v.at[j]], sem, add=True)
            return carry

        lax.fori_loop(0, nch, fire, 0)

        def drain(j, carry):
            pltpu.make_async_copy(ones_v, acc.at[dst_v.at[j]], sem).wait()
            return carry

        lax.fori_loop(0, nch, drain, 0)
        plsc.subcore_barrier()
        pltpu.sync_copy(
            acc.at[pl.ds(s * rows_per_tile, rows_per_tile)],
            out_hbm.at[c].at[pl.ds(s * rows_per_tile, rows_per_tile)])

    return deg_k


@functools.lru_cache(maxsize=None)
def _sc_agg(n_pad: int, d: int, nch: int):
    dh = d // 2
    rows_per_tile = n_pad // NS
    zcopies = rows_per_tile // CHUNK
    nb2 = 2 * NBUF

    scratch = [
        pltpu.VMEM((nch, CHUNK), jnp.int32),
        pltpu.VMEM((nch, CHUNK), jnp.int32),
        pltpu.VMEM((nb2, CHUNK, dh), jnp.float32),
        pltpu.VMEM((CHUNK, dh), jnp.float32),
        pltpu.VMEM_SHARED((n_pad, dh), jnp.float32),
    ] + [pltpu.SemaphoreType.DMA] * (2 * nb2)

    @functools.partial(
        pl.kernel,
        out_type=jax.ShapeDtypeStruct((NC, 2, n_pad, dh), jnp.float32),
        mesh=_mesh(),
        scratch_types=scratch,
        compiler_params=pltpu.CompilerParams(use_tc_tiling_on_sc=False),
    )
    def agg_k(xw_hbm, src_hbm, dst_hbm, out_hbm, src_v, dst_v, buf, zbuf, acc,
              *sems):
        gsems = sems[:nb2]
        ssems = sems[nb2:]
        c = lax.axis_index("c")
        s = lax.axis_index("s")
        wid = s * NC + c
        pltpu.sync_copy(src_hbm.at[wid], src_v)
        pltpu.sync_copy(dst_hbm.at[wid], dst_v)

        zeros = jnp.zeros((LANES,), jnp.float32)

        def zbody(i, carry):
            for k in range(dh // LANES):
                zbuf[i, pl.ds(k * LANES, LANES)] = zeros
            return carry

        lax.fori_loop(0, CHUNK, zbody, 0)

        for half in range(2):
            xw_h = xw_hbm.at[half]
            for k in range(zcopies):
                pltpu.sync_copy(
                    zbuf, acc.at[pl.ds(s * rows_per_tile + k * CHUNK, CHUNK)])
            plsc.subcore_barrier()

            for b in range(NBUF):
                pltpu.async_copy(xw_h.at[src_v.at[b]], buf.at[b], gsems[b])

            def step(jo, carry):
                for u in range(nb2):
                    j = jo * nb2 + u
                    b = u
                    pltpu.make_async_copy(
                        xw_h.at[src_v.at[j]], buf.at[b], gsems[b]).wait()
                    pltpu.async_copy(
                        buf.at[b], acc.at[dst_v.at[j]], ssems[b], add=True)
                    jn = j + NBUF
                    bn = (u + NBUF) % nb2

                    @pl.when(jn < nch)
                    def _():
                        @pl.when(jn >= nb2)
                        def _():
                            pltpu.make_async_copy(
                                buf.at[bn], acc.at[dst_v.at[jn]],
                                ssems[bn]).wait()

                        pltpu.async_copy(
                            xw_h.at[src_v.at[jn]], buf.at[bn], gsems[bn])

                return carry

            lax.fori_loop(0, nch // nb2, step, 0)

            for u in range(nb2):
                j = nch - nb2 + u
                pltpu.make_async_copy(
                    buf.at[u], acc.at[dst_v.at[j]], ssems[u]).wait()

            plsc.subcore_barrier()
            pltpu.sync_copy(
                acc.at[pl.ds(s * rows_per_tile, rows_per_tile)],
                out_hbm.at[c].at[half].at[pl.ds(s * rows_per_tile,
                                                rows_per_tile)])
            plsc.subcore_barrier()

    return agg_k


def _dis_block(dp_ref):
    deg = dp_ref[0, :, 0:1] + dp_ref[1, :, 0:1] + 1.0
    return lax.rsqrt(deg)


def _deg_spec():
    return pl.BlockSpec((NC, ROWBLK, LANES), lambda i: (0, i, 0))


def _tc_scale_matmul(x_pad, W, deg_parts):
    n_pad, d = x_pad.shape
    dh = d // 2

    def body(x_ref, w_ref, dp_ref, o_ref):
        dis = _dis_block(dp_ref)
        xw = jnp.dot(x_ref[...], w_ref[0],
                     precision=lax.Precision.HIGHEST,
                     preferred_element_type=jnp.float32)
        o_ref[0] = xw * dis

    return pl.pallas_call(
        body,
        grid=(2, n_pad // ROWBLK),
        in_specs=[
            pl.BlockSpec((ROWBLK, d), lambda j, i: (i, 0)),
            pl.BlockSpec((1, d, dh), lambda j, i: (j, 0, 0)),
            pl.BlockSpec((NC, ROWBLK, LANES), lambda j, i: (0, i, 0)),
        ],
        out_specs=pl.BlockSpec((1, ROWBLK, dh), lambda j, i: (j, i, 0)),
        out_shape=jax.ShapeDtypeStruct((2, n_pad, dh), jnp.float32),
    )(x_pad, W, deg_parts)


def _agg_block(p_ref, xw_ref, dis):
    return jnp.concatenate(
        [(p_ref[0, h] + p_ref[1, h] + xw_ref[h]) * dis for h in range(2)],
        axis=1)


def _tc_mid(parts, xw_s, deg_parts, b, W):
    _, n_pad, dh = xw_s.shape
    d = 2 * dh

    def body(p_ref, xw_ref, dp_ref, b_ref, w_ref, o_ref):
        dis = _dis_block(dp_ref)
        h = jnp.maximum(_agg_block(p_ref, xw_ref, dis) + b_ref[...], 0.0)
        o_ref[0] = jnp.dot(h, w_ref[0],
                           precision=lax.Precision.HIGHEST,
                           preferred_element_type=jnp.float32) * dis

    return pl.pallas_call(
        body,
        grid=(2, n_pad // ROWBLK),
        in_specs=[
            pl.BlockSpec((NC, 2, ROWBLK, dh), lambda j, i: (0, 0, i, 0)),
            pl.BlockSpec((2, ROWBLK, dh), lambda j, i: (0, i, 0)),
            pl.BlockSpec((NC, ROWBLK, LANES), lambda j, i: (0, i, 0)),
            pl.BlockSpec((1, d), lambda j, i: (0, 0)),
            pl.BlockSpec((1, d, dh), lambda j, i: (j, 0, 0)),
        ],
        out_specs=pl.BlockSpec((1, ROWBLK, dh), lambda j, i: (j, i, 0)),
        out_shape=jax.ShapeDtypeStruct((2, n_pad, dh), jnp.float32),
    )(parts, xw_s, deg_parts, b, W)


def _tc_final(parts, xw_s, deg_parts, b):
    _, n_pad, dh = xw_s.shape
    d = 2 * dh

    def body(p_ref, xw_ref, dp_ref, b_ref, o_ref):
        dis = _dis_block(dp_ref)
        o_ref[...] = _agg_block(p_ref, xw_ref, dis) + b_ref[...]

    return pl.pallas_call(
        body,
        grid=(n_pad // ROWBLK,),
        in_specs=[
            pl.BlockSpec((NC, 2, ROWBLK, dh), lambda i: (0, 0, i, 0)),
            pl.BlockSpec((2, ROWBLK, dh), lambda i: (0, i, 0)),
            _deg_spec(),
            pl.BlockSpec((1, d), lambda i: (0, 0)),
        ],
        out_specs=pl.BlockSpec((ROWBLK, d), lambda i: (i, 0)),
        out_shape=jax.ShapeDtypeStruct((n_pad, d), jnp.float32),
    )(parts, xw_s, deg_parts, b)


def kernel(x, edge_index, W1, b1, W2, b2):
    n, d = x.shape
    e = edge_index.shape[1]

    n_pad = -(-(n + 1) // ROWBLK) * ROWBLK
    epw = -(-e // NW)
    nch = -(-epw // CHUNK)
    nch = -(-nch // (2 * NBUF)) * 2 * NBUF
    e_pad = NW * nch * CHUNK

    npad_e = e_pad - e
    ramp = jnp.arange(npad_e, dtype=edge_index.dtype)
    pad_src = n + ramp % (n_pad - n)
    pad_dst_agg = ramp % n_pad
    pad_dst_deg = n + ramp % (n_pad - n)
    src = jnp.concatenate([edge_index[0], pad_src]).reshape(NW, nch, CHUNK)
    dst = jnp.concatenate([edge_index[1], pad_dst_agg]).reshape(NW, nch, CHUNK)
    dstd = jnp.concatenate([edge_index[1], pad_dst_deg]).reshape(NW, nch, CHUNK)

    x_pad = jnp.zeros((n_pad, d), jnp.float32).at[:n].set(x)

    deg_parts = _sc_degree(n_pad, nch)(dstd)

    b1r = b1.reshape(1, d)
    b2r = b2.reshape(1, d)
    dh = d // 2
    W1h = jnp.stack([W1[:, :dh], W1[:, dh:]])
    W2h = jnp.stack([W2[:, :dh], W2[:, dh:]])

    xw1s = _tc_scale_matmul(x_pad, W1h, deg_parts)
    parts1 = _sc_agg(n_pad, d, nch)(xw1s, src, dst)
    xw2s = _tc_mid(parts1, xw1s, deg_parts, b1r, W2h)
    parts2 = _sc_agg(n_pad, d, nch)(xw2s, src, dst)
    out = _tc_final(parts2, xw2s, deg_parts, b2r)
    return out[:n]


# speedup vs baseline: 29.8484x; 1.2652x over previous
import functools

import jax
import jax.numpy as jnp
from jax import lax
from jax.experimental import pallas as pl
from jax.experimental.pallas import tpu as pltpu
from jax.experimental.pallas import tpu_sc as plsc

NC = 2
NS = 16
NW = NC * NS
LANES = 16
CHUNK = 128
NBUF = 2
ROWBLK = 1280


def _mesh():
    return plsc.VectorSubcoreMesh(core_axis_name="c", subcore_axis_name="s")


@functools.lru_cache(maxsize=None)
def _sc_degree(n_pad: int, nch: int):
    rows_per_tile = n_pad // NS
    zcopies = rows_per_tile // CHUNK

    scratch = [
        pltpu.VMEM((nch, CHUNK), jnp.int32),
        pltpu.VMEM((CHUNK, LANES), jnp.float32),
        pltpu.VMEM((CHUNK, LANES), jnp.float32),
        pltpu.VMEM_SHARED((n_pad, LANES), jnp.float32),
        pltpu.SemaphoreType.DMA,
    ]

    @functools.partial(
        pl.kernel,
        out_type=jax.ShapeDtypeStruct((NC, n_pad, LANES), jnp.float32),
        mesh=_mesh(),
        scratch_types=scratch,
        compiler_params=pltpu.CompilerParams(use_tc_tiling_on_sc=False),
    )
    def deg_k(dst_hbm, out_hbm, dst_v, ones_v, zero_v, acc, sem):
        c = lax.axis_index("c")
        s = lax.axis_index("s")
        wid = s * NC + c
        pltpu.sync_copy(dst_hbm.at[wid], dst_v)

        ones = jnp.ones((LANES,), jnp.float32)
        zeros = jnp.zeros((LANES,), jnp.float32)

        def fbody(i, carry):
            ones_v[i, :] = ones
            zero_v[i, :] = zeros
            return carry

        lax.fori_loop(0, CHUNK, fbody, 0)
        for k in range(zcopies):
            pltpu.sync_copy(
                zero_v, acc.at[pl.ds(s * rows_per_tile + k * CHUNK, CHUNK)])
        plsc.subcore_barrier()

        def fire(j, carry):
            pltpu.async_copy(ones_v, acc.at[dst_v.at[j]], sem, add=True)
            return carry

        lax.fori_loop(0, nch, fire, 0)

        def drain(j, carry):
            pltpu.make_async_copy(ones_v, acc.at[dst_v.at[j]], sem).wait()
            return carry

        lax.fori_loop(0, nch, drain, 0)
        plsc.subcore_barrier()
        pltpu.sync_copy(
            acc.at[pl.ds(s * rows_per_tile, rows_per_tile)],
            out_hbm.at[c].at[pl.ds(s * rows_per_tile, rows_per_tile)])

    return deg_k


@functools.lru_cache(maxsize=None)
def _sc_agg(n_pad: int, d: int, nch: int):
    dh = d // 2
    rows_per_tile = n_pad // NS
    zcopies = rows_per_tile // CHUNK
    nb2 = 2 * NBUF

    scratch = [
        pltpu.VMEM((nch, CHUNK), jnp.int32),
        pltpu.VMEM((nch, CHUNK), jnp.int32),
        pltpu.VMEM((nch, CHUNK), jnp.int32),
        pltpu.VMEM((nb2, CHUNK, dh), jnp.float32),
        pltpu.VMEM((CHUNK, dh), jnp.float32),
        pltpu.VMEM_SHARED((n_pad, dh), jnp.float32),
    ] + [pltpu.SemaphoreType.DMA] * (2 * nb2)

    @functools.partial(
        pl.kernel,
        out_type=jax.ShapeDtypeStruct((NC, n_pad, d), jnp.float32),
        mesh=_mesh(),
        scratch_types=scratch,
        compiler_params=pltpu.CompilerParams(use_tc_tiling_on_sc=False),
    )
    def agg_k(xw_hbm, srca_hbm, srcb_hbm, dst_hbm, out_hbm, srca_v, srcb_v,
              dst_v, buf, zbuf, acc, *sems):
        gsems = sems[:nb2]
        ssems = sems[nb2:]
        c = lax.axis_index("c")
        s = lax.axis_index("s")
        wid = s * NC + c
        pltpu.sync_copy(srca_hbm.at[wid], srca_v)
        pltpu.sync_copy(srcb_hbm.at[wid], srcb_v)
        pltpu.sync_copy(dst_hbm.at[wid], dst_v)

        zeros = jnp.zeros((LANES,), jnp.float32)

        def zbody(i, carry):
            for k in range(dh // LANES):
                zbuf[i, pl.ds(k * LANES, LANES)] = zeros
            return carry

        lax.fori_loop(0, CHUNK, zbody, 0)

        for half in range(2):
            src_v = srca_v if half == 0 else srcb_v
            for k in range(zcopies):
                pltpu.sync_copy(
                    zbuf, acc.at[pl.ds(s * rows_per_tile + k * CHUNK, CHUNK)])
            plsc.subcore_barrier()

            for b in range(NBUF):
                pltpu.async_copy(xw_hbm.at[src_v.at[b]], buf.at[b], gsems[b])

            def step(jo, carry):
                for u in range(nb2):
                    j = jo * nb2 + u
                    b = u
                    pltpu.make_async_copy(
                        xw_hbm.at[src_v.at[j]], buf.at[b], gsems[b]).wait()
                    pltpu.async_copy(
                        buf.at[b], acc.at[dst_v.at[j]], ssems[b], add=True)
                    jn = j + NBUF
                    bn = (u + NBUF) % nb2

                    @pl.when(jn < nch)
                    def _():
                        @pl.when(jn >= nb2)
                        def _():
                            pltpu.make_async_copy(
                                buf.at[bn], acc.at[dst_v.at[jn]],
                                ssems[bn]).wait()

                        pltpu.async_copy(
                            xw_hbm.at[src_v.at[jn]], buf.at[bn], gsems[bn])

                return carry

            lax.fori_loop(0, nch // nb2, step, 0)

            for u in range(nb2):
                j = nch - nb2 + u
                pltpu.make_async_copy(
                    buf.at[u], acc.at[dst_v.at[j]], ssems[u]).wait()

            plsc.subcore_barrier()
            pltpu.sync_copy(
                acc.at[pl.ds(s * rows_per_tile, rows_per_tile)],
                out_hbm.at[c].at[pl.ds(s * rows_per_tile, rows_per_tile),
                                 pl.ds(half * dh, dh)])
            plsc.subcore_barrier()

    return agg_k


def _dis_block(dp_ref):
    deg = dp_ref[0, :, 0:1] + dp_ref[1, :, 0:1] + 1.0
    return lax.rsqrt(deg)


def _deg_spec():
    return pl.BlockSpec((NC, ROWBLK, LANES), lambda i: (0, i, 0))


def _tc_scale_matmul(x_pad, W, deg_parts):
    n_pad, d = x_pad.shape

    def body(x_ref, w_ref, dp_ref, o_ref):
        dis = _dis_block(dp_ref)
        o_ref[...] = jnp.dot(x_ref[...], w_ref[...],
                             precision=lax.Precision.HIGHEST,
                             preferred_element_type=jnp.float32) * dis

    return pl.pallas_call(
        body,
        grid=(n_pad // ROWBLK,),
        in_specs=[
            pl.BlockSpec((ROWBLK, d), lambda i: (i, 0)),
            pl.BlockSpec((d, d), lambda i: (0, 0)),
            _deg_spec(),
        ],
        out_specs=pl.BlockSpec((ROWBLK, d), lambda i: (i, 0)),
        out_shape=jax.ShapeDtypeStruct((n_pad, d), jnp.float32),
    )(x_pad, W, deg_parts)


def _agg_block(p_ref, xw_ref):
    return p_ref[0] + p_ref[1] + xw_ref[...]


def _tc_mid(parts, xw_s, deg_parts, b, W):
    n_pad, d = xw_s.shape

    def body(p_ref, xw_ref, dp_ref, b_ref, w_ref, o_ref):
        dis = _dis_block(dp_ref)
        h = jnp.maximum(_agg_block(p_ref, xw_ref) * dis + b_ref[...], 0.0)
        o_ref[...] = jnp.dot(h, w_ref[...],
                             precision=lax.Precision.HIGHEST,
                             preferred_element_type=jnp.float32) * dis

    return pl.pallas_call(
        body,
        grid=(n_pad // ROWBLK,),
        in_specs=[
            pl.BlockSpec((NC, ROWBLK, d), lambda i: (0, i, 0)),
            pl.BlockSpec((ROWBLK, d), lambda i: (i, 0)),
            _deg_spec(),
            pl.BlockSpec((1, d), lambda i: (0, 0)),
            pl.BlockSpec((d, d), lambda i: (0, 0)),
        ],
        out_specs=pl.BlockSpec((ROWBLK, d), lambda i: (i, 0)),
        out_shape=jax.ShapeDtypeStruct((n_pad, d), jnp.float32),
    )(parts, xw_s, deg_parts, b, W)


def _tc_final(parts, xw_s, deg_parts, b, n):
    n_pad, d = xw_s.shape

    def body(p_ref, xw_ref, dp_ref, b_ref, o_ref):
        dis = _dis_block(dp_ref)
        o_ref[...] = _agg_block(p_ref, xw_ref) * dis + b_ref[...]

    return pl.pallas_call(
        body,
        grid=(n_pad // ROWBLK,),
        in_specs=[
            pl.BlockSpec((NC, ROWBLK, d), lambda i: (0, i, 0)),
            pl.BlockSpec((ROWBLK, d), lambda i: (i, 0)),
            _deg_spec(),
            pl.BlockSpec((1, d), lambda i: (0, 0)),
        ],
        out_specs=pl.BlockSpec((ROWBLK, d), lambda i: (i, 0)),
        out_shape=jax.ShapeDtypeStruct((n, d), jnp.float32),
    )(parts, xw_s, deg_parts, b)


def kernel(x, edge_index, W1, b1, W2, b2):
    n, d = x.shape
    e = edge_index.shape[1]

    n_pad = -(-(n + 1) // ROWBLK) * ROWBLK
    epw = -(-e // NW)
    nch = -(-epw // CHUNK)
    nch = -(-nch // (2 * NBUF)) * 2 * NBUF
    e_pad = NW * nch * CHUNK

    npad_e = e_pad - e
    ramp = jnp.arange(npad_e, dtype=edge_index.dtype)
    pad_src = n + ramp % (n_pad - n)
    pad_dst_agg = ramp % n_pad
    pad_dst_deg = n + ramp % (n_pad - n)
    src = jnp.concatenate([edge_index[0], pad_src]).reshape(NW, nch, CHUNK)
    dst = jnp.concatenate([edge_index[1], pad_dst_agg]).reshape(NW, nch, CHUNK)
    dstd = jnp.concatenate([edge_index[1], pad_dst_deg]).reshape(NW, nch, CHUNK)

    x_pad = jnp.zeros((n_pad, d), jnp.float32).at[:n].set(x)

    deg_parts = _sc_degree(n_pad, nch)(dstd)

    b1r = b1.reshape(1, d)
    b2r = b2.reshape(1, d)
    dh = d // 2
    srca = src * 2
    srcb = srca + 1

    agg = _sc_agg(n_pad, d, nch)

    xw1s = _tc_scale_matmul(x_pad, W1, deg_parts)
    parts1 = agg(xw1s.reshape(2 * n_pad, dh), srca, srcb, dst)
    xw2s = _tc_mid(parts1, xw1s, deg_parts, b1r, W2)
    parts2 = agg(xw2s.reshape(2 * n_pad, dh), srca, srcb, dst)
    return _tc_final(parts2, xw2s, deg_parts, b2r, n)


# speedup vs baseline: 30.2633x; 1.0139x over previous
import functools

import jax
import jax.numpy as jnp
from jax import lax
from jax.experimental import pallas as pl
from jax.experimental.pallas import tpu as pltpu
from jax.experimental.pallas import tpu_sc as plsc

NC = 2
NS = 16
NW = NC * NS
LANES = 16
CHUNK = 128
NBUF = 2
ROWBLK = 1280


def _mesh():
    return plsc.VectorSubcoreMesh(core_axis_name="c", subcore_axis_name="s")


@functools.lru_cache(maxsize=None)
def _sc_degree(n_pad: int, nch: int):
    rows_per_tile = n_pad // NS
    zcopies = rows_per_tile // CHUNK

    scratch = [
        pltpu.VMEM((nch, CHUNK), jnp.int32),
        pltpu.VMEM((CHUNK, LANES), jnp.float32),
        pltpu.VMEM((CHUNK, LANES), jnp.float32),
        pltpu.VMEM_SHARED((n_pad, LANES), jnp.float32),
        pltpu.SemaphoreType.DMA,
    ]

    @functools.partial(
        pl.kernel,
        out_type=jax.ShapeDtypeStruct((NC, n_pad, LANES), jnp.float32),
        mesh=_mesh(),
        scratch_types=scratch,
        compiler_params=pltpu.CompilerParams(use_tc_tiling_on_sc=False),
    )
    def deg_k(dst_hbm, out_hbm, dst_v, ones_v, zero_v, acc, sem):
        c = lax.axis_index("c")
        s = lax.axis_index("s")
        wid = s * NC + c
        pltpu.sync_copy(dst_hbm.at[wid], dst_v)

        ones = jnp.ones((LANES,), jnp.float32)
        zeros = jnp.zeros((LANES,), jnp.float32)

        def fbody(i, carry):
            ones_v[i, :] = ones
            zero_v[i, :] = zeros
            return carry

        lax.fori_loop(0, CHUNK, fbody, 0)
        for k in range(zcopies):
            pltpu.sync_copy(
                zero_v, acc.at[pl.ds(s * rows_per_tile + k * CHUNK, CHUNK)])
        plsc.subcore_barrier()

        def fire(j, carry):
            pltpu.async_copy(ones_v, acc.at[dst_v.at[j]], sem, add=True)
            return carry

        lax.fori_loop(0, nch, fire, 0)

        def drain(j, carry):
            pltpu.make_async_copy(ones_v, acc.at[dst_v.at[j]], sem).wait()
            return carry

        lax.fori_loop(0, nch, drain, 0)
        plsc.subcore_barrier()
        pltpu.sync_copy(
            acc.at[pl.ds(s * rows_per_tile, rows_per_tile)],
            out_hbm.at[c].at[pl.ds(s * rows_per_tile, rows_per_tile)])

    return deg_k


@functools.lru_cache(maxsize=None)
def _sc_agg(n_pad: int, d: int, nch: int):
    dh = d // 2
    rows_per_tile = n_pad // NS
    zcopies = rows_per_tile // CHUNK
    nb2 = 2 * NBUF

    scratch = [
        pltpu.VMEM((nch, CHUNK), jnp.int32),
        pltpu.VMEM((nch, CHUNK), jnp.int32),
        pltpu.VMEM((nch, CHUNK), jnp.int32),
        pltpu.VMEM((nb2, CHUNK, dh), jnp.float32),
        pltpu.VMEM((CHUNK, dh), jnp.float32),
        pltpu.VMEM_SHARED((n_pad, dh), jnp.float32),
    ] + [pltpu.SemaphoreType.DMA] * (2 * nb2)

    @functools.partial(
        pl.kernel,
        out_type=jax.ShapeDtypeStruct((NC, n_pad, d), jnp.float32),
        mesh=_mesh(),
        scratch_types=scratch,
        compiler_params=pltpu.CompilerParams(use_tc_tiling_on_sc=False),
    )
    def agg_k(xw_hbm, srca_hbm, srcb_hbm, dst_hbm, out_hbm, srca_v, srcb_v,
              dst_v, buf, zbuf, acc, *sems):
        gsems = sems[:nb2]
        ssems = sems[nb2:]
        c = lax.axis_index("c")
        s = lax.axis_index("s")
        wid = s * NC + c
        pltpu.sync_copy(srca_hbm.at[wid], srca_v)
        pltpu.sync_copy(srcb_hbm.at[wid], srcb_v)
        pltpu.sync_copy(dst_hbm.at[wid], dst_v)

        zeros = jnp.zeros((LANES,), jnp.float32)

        def zbody(i, carry):
            for k in range(dh // LANES):
                zbuf[i, pl.ds(k * LANES, LANES)] = zeros
            return carry

        lax.fori_loop(0, CHUNK, zbody, 0)

        for b in range(NBUF):
            pltpu.async_copy(xw_hbm.at[srca_v.at[b]], buf.at[b], gsems[b])

        for half in range(2):
            src_v = srca_v if half == 0 else srcb_v
            for k in range(zcopies):
                pltpu.sync_copy(
                    zbuf, acc.at[pl.ds(s * rows_per_tile + k * CHUNK, CHUNK)])
            plsc.subcore_barrier()

            def step(jo, carry):
                for u in range(nb2):
                    j = jo * nb2 + u
                    b = u
                    pltpu.make_async_copy(
                        xw_hbm.at[src_v.at[j]], buf.at[b], gsems[b]).wait()
                    pltpu.async_copy(
                        buf.at[b], acc.at[dst_v.at[j]], ssems[b], add=True)
                    jn = j + NBUF
                    bn = (u + NBUF) % nb2

                    @pl.when(jn < nch)
                    def _():
                        @pl.when(jn >= nb2)
                        def _():
                            pltpu.make_async_copy(
                                buf.at[bn], acc.at[dst_v.at[jn]],
                                ssems[bn]).wait()

                        pltpu.async_copy(
                            xw_hbm.at[src_v.at[jn]], buf.at[bn], gsems[bn])

                return carry

            lax.fori_loop(0, nch // nb2, step, 0)

            for u in range(nb2):
                j = nch - nb2 + u
                pltpu.make_async_copy(
                    buf.at[u], acc.at[dst_v.at[j]], ssems[u]).wait()

            if half == 0:
                for b in range(NBUF):
                    pltpu.async_copy(
                        xw_hbm.at[srcb_v.at[b]], buf.at[b], gsems[b])

            plsc.subcore_barrier()
            pltpu.sync_copy(
                acc.at[pl.ds(s * rows_per_tile, rows_per_tile)],
                out_hbm.at[c].at[pl.ds(s * rows_per_tile, rows_per_tile),
                                 pl.ds(half * dh, dh)])

    return agg_k


def _dis_block(dp_ref):
    deg = dp_ref[0, :, 0:1] + dp_ref[1, :, 0:1] + 1.0
    return lax.rsqrt(deg)


def _deg_spec():
    return pl.BlockSpec((NC, ROWBLK, LANES), lambda i: (0, i, 0))


def _tc_scale_matmul(x_pad, W, deg_parts):
    n_pad, d = x_pad.shape

    def body(x_ref, w_ref, dp_ref, o_ref):
        dis = _dis_block(dp_ref)
        o_ref[...] = jnp.dot(x_ref[...], w_ref[...],
                             precision=lax.Precision.HIGHEST,
                             preferred_element_type=jnp.float32) * dis

    return pl.pallas_call(
        body,
        grid=(n_pad // ROWBLK,),
        in_specs=[
            pl.BlockSpec((ROWBLK, d), lambda i: (i, 0)),
            pl.BlockSpec((d, d), lambda i: (0, 0)),
            _deg_spec(),
        ],
        out_specs=pl.BlockSpec((ROWBLK, d), lambda i: (i, 0)),
        out_shape=jax.ShapeDtypeStruct((n_pad, d), jnp.float32),
    )(x_pad, W, deg_parts)


def _agg_block(p_ref, xw_ref):
    return p_ref[0] + p_ref[1] + xw_ref[...]


def _tc_mid(parts, xw_s, deg_parts, b, W):
    n_pad, d = xw_s.shape

    def body(p_ref, xw_ref, dp_ref, b_ref, w_ref, o_ref):
        dis = _dis_block(dp_ref)
        h = jnp.maximum(_agg_block(p_ref, xw_ref) * dis + b_ref[...], 0.0)
        o_ref[...] = jnp.dot(h, w_ref[...],
                             precision=lax.Precision.HIGHEST,
                             preferred_element_type=jnp.float32) * dis

    return pl.pallas_call(
        body,
        grid=(n_pad // ROWBLK,),
        in_specs=[
            pl.BlockSpec((NC, ROWBLK, d), lambda i: (0, i, 0)),
            pl.BlockSpec((ROWBLK, d), lambda i: (i, 0)),
            _deg_spec(),
            pl.BlockSpec((1, d), lambda i: (0, 0)),
            pl.BlockSpec((d, d), lambda i: (0, 0)),
        ],
        out_specs=pl.BlockSpec((ROWBLK, d), lambda i: (i, 0)),
        out_shape=jax.ShapeDtypeStruct((n_pad, d), jnp.float32),
    )(parts, xw_s, deg_parts, b, W)


def _tc_final(parts, xw_s, deg_parts, b, n):
    n_pad, d = xw_s.shape

    def body(p_ref, xw_ref, dp_ref, b_ref, o_ref):
        dis = _dis_block(dp_ref)
        o_ref[...] = _agg_block(p_ref, xw_ref) * dis + b_ref[...]

    return pl.pallas_call(
        body,
        grid=(n_pad // ROWBLK,),
        in_specs=[
            pl.BlockSpec((NC, ROWBLK, d), lambda i: (0, i, 0)),
            pl.BlockSpec((ROWBLK, d), lambda i: (i, 0)),
            _deg_spec(),
            pl.BlockSpec((1, d), lambda i: (0, 0)),
        ],
        out_specs=pl.BlockSpec((ROWBLK, d), lambda i: (i, 0)),
        out_shape=jax.ShapeDtypeStruct((n, d), jnp.float32),
    )(parts, xw_s, deg_parts, b)


def kernel(x, edge_index, W1, b1, W2, b2):
    n, d = x.shape
    e = edge_index.shape[1]

    n_pad = -(-(n + 1) // ROWBLK) * ROWBLK
    epw = -(-e // NW)
    nch = -(-epw // CHUNK)
    nch = -(-nch // (2 * NBUF)) * 2 * NBUF
    e_pad = NW * nch * CHUNK

    npad_e = e_pad - e
    ramp = jnp.arange(npad_e, dtype=edge_index.dtype)
    pad_src = n + ramp % (n_pad - n)
    pad_dst_agg = ramp % n_pad
    pad_dst_deg = n + ramp % (n_pad - n)
    src = jnp.concatenate([edge_index[0], pad_src]).reshape(NW, nch, CHUNK)
    dst = jnp.concatenate([edge_index[1], pad_dst_agg]).reshape(NW, nch, CHUNK)
    dstd = jnp.concatenate([edge_index[1], pad_dst_deg]).reshape(NW, nch, CHUNK)

    x_pad = jnp.zeros((n_pad, d), jnp.float32).at[:n].set(x)

    deg_parts = _sc_degree(n_pad, nch)(dstd)

    b1r = b1.reshape(1, d)
    b2r = b2.reshape(1, d)
    dh = d // 2
    srca = src * 2
    srcb = srca + 1

    agg = _sc_agg(n_pad, d, nch)

    xw1s = _tc_scale_matmul(x_pad, W1, deg_parts)
    parts1 = agg(xw1s.reshape(2 * n_pad, dh), srca, srcb, dst)
    xw2s = _tc_mid(parts1, xw1s, deg_parts, b1r, W2)
    parts2 = agg(xw2s.reshape(2 * n_pad, dh), srca, srcb, dst)
    return _tc_final(parts2, xw2s, deg_parts, b2r, n)


# speedup vs baseline: 30.6101x; 1.0115x over previous
import functools

import jax
import jax.numpy as jnp
from jax import lax
from jax.experimental import pallas as pl
from jax.experimental.pallas import tpu as pltpu
from jax.experimental.pallas import tpu_sc as plsc

NC = 2
NS = 16
NW = NC * NS
LANES = 16
CHUNK = 128
NBUF = 2
ROWBLK = 1280


def _mesh():
    return plsc.VectorSubcoreMesh(core_axis_name="c", subcore_axis_name="s")


@functools.lru_cache(maxsize=None)
def _sc_degree(n_pad: int, nch: int):
    rows_per_tile = n_pad // NS
    zcopies = rows_per_tile // CHUNK

    scratch = [
        pltpu.VMEM((nch, CHUNK), jnp.int32),
        pltpu.VMEM((CHUNK, LANES), jnp.float32),
        pltpu.VMEM((CHUNK, LANES), jnp.float32),
        pltpu.VMEM_SHARED((n_pad, LANES), jnp.float32),
        pltpu.SemaphoreType.DMA,
    ]

    @functools.partial(
        pl.kernel,
        out_type=jax.ShapeDtypeStruct((NC, n_pad, 128), jnp.float32),
        mesh=_mesh(),
        scratch_types=scratch,
        compiler_params=pltpu.CompilerParams(use_tc_tiling_on_sc=False),
    )
    def deg_k(dst_hbm, out_hbm, dst_v, ones_v, zero_v, acc, sem):
        c = lax.axis_index("c")
        s = lax.axis_index("s")
        wid = s * NC + c
        pltpu.sync_copy(dst_hbm.at[wid], dst_v)

        ones = jnp.ones((LANES,), jnp.float32)
        zeros = jnp.zeros((LANES,), jnp.float32)

        def fbody(i, carry):
            ones_v[i, :] = ones
            zero_v[i, :] = zeros
            return carry

        lax.fori_loop(0, CHUNK, fbody, 0)
        for k in range(zcopies):
            pltpu.sync_copy(
                zero_v, acc.at[pl.ds(s * rows_per_tile + k * CHUNK, CHUNK)])
        plsc.subcore_barrier()

        def fire(j, carry):
            pltpu.async_copy(ones_v, acc.at[dst_v.at[j]], sem, add=True)
            return carry

        lax.fori_loop(0, nch, fire, 0)

        def drain(j, carry):
            pltpu.make_async_copy(ones_v, acc.at[dst_v.at[j]], sem).wait()
            return carry

        lax.fori_loop(0, nch, drain, 0)
        plsc.subcore_barrier()
        pltpu.sync_copy(
            acc.at[pl.ds(s * rows_per_tile, rows_per_tile)],
            out_hbm.at[c].at[pl.ds(s * rows_per_tile, rows_per_tile),
                             pl.ds(0, LANES)])

    return deg_k


@functools.lru_cache(maxsize=None)
def _sc_agg(n_pad: int, d: int, nch: int):
    dh = d // 2
    rows_per_tile = n_pad // NS
    zcopies = rows_per_tile // CHUNK
    nb2 = 2 * NBUF

    scratch = [
        pltpu.VMEM((nch, CHUNK), jnp.int32),
        pltpu.VMEM((nch, CHUNK), jnp.int32),
        pltpu.VMEM((nch, CHUNK), jnp.int32),
        pltpu.VMEM((nb2, CHUNK, dh), jnp.float32),
        pltpu.VMEM((CHUNK, dh), jnp.float32),
        pltpu.VMEM_SHARED((n_pad, dh), jnp.float32),
    ] + [pltpu.SemaphoreType.DMA] * (2 * nb2)

    @functools.partial(
        pl.kernel,
        out_type=jax.ShapeDtypeStruct((NC, n_pad, d), jnp.float32),
        mesh=_mesh(),
        scratch_types=scratch,
        compiler_params=pltpu.CompilerParams(use_tc_tiling_on_sc=False),
    )
    def agg_k(xw_hbm, srca_hbm, srcb_hbm, dst_hbm, out_hbm, srca_v, srcb_v,
              dst_v, buf, zbuf, acc, *sems):
        gsems = sems[:nb2]
        ssems = sems[nb2:]
        c = lax.axis_index("c")
        s = lax.axis_index("s")
        wid = s * NC + c
        pltpu.sync_copy(srca_hbm.at[wid], srca_v)
        pltpu.sync_copy(srcb_hbm.at[wid], srcb_v)
        pltpu.sync_copy(dst_hbm.at[wid], dst_v)

        zeros = jnp.zeros((LANES,), jnp.float32)

        def zbody(i, carry):
            for k in range(dh // LANES):
                zbuf[i, pl.ds(k * LANES, LANES)] = zeros
            return carry

        lax.fori_loop(0, CHUNK, zbody, 0)

        for b in range(NBUF):
            pltpu.async_copy(xw_hbm.at[srca_v.at[b]], buf.at[b], gsems[b])

        for half in range(2):
            src_v = srca_v if half == 0 else srcb_v
            for k in range(zcopies):
                pltpu.sync_copy(
                    zbuf, acc.at[pl.ds(s * rows_per_tile + k * CHUNK, CHUNK)])
            plsc.subcore_barrier()

            def step(jo, carry):
                for u in range(nb2):
                    j = jo * nb2 + u
                    b = u
                    pltpu.make_async_copy(
                        xw_hbm.at[src_v.at[j]], buf.at[b], gsems[b]).wait()
                    pltpu.async_copy(
                        buf.at[b], acc.at[dst_v.at[j]], ssems[b], add=True)
                    jn = j + NBUF
                    bn = (u + NBUF) % nb2

                    @pl.when(jn < nch)
                    def _():
                        @pl.when(jn >= nb2)
                        def _():
                            pltpu.make_async_copy(
                                buf.at[bn], acc.at[dst_v.at[jn]],
                                ssems[bn]).wait()

                        pltpu.async_copy(
                            xw_hbm.at[src_v.at[jn]], buf.at[bn], gsems[bn])

                return carry

            lax.fori_loop(0, nch // nb2, step, 0)

            for u in range(nb2):
                j = nch - nb2 + u
                pltpu.make_async_copy(
                    buf.at[u], acc.at[dst_v.at[j]], ssems[u]).wait()

            if half == 0:
                for b in range(NBUF):
                    pltpu.async_copy(
                        xw_hbm.at[srcb_v.at[b]], buf.at[b], gsems[b])

            plsc.subcore_barrier()
            pltpu.sync_copy(
                acc.at[pl.ds(s * rows_per_tile, rows_per_tile)],
                out_hbm.at[c].at[pl.ds(s * rows_per_tile, rows_per_tile),
                                 pl.ds(half * dh, dh)])

    return agg_k


def _dis_block(dp_ref):
    deg = dp_ref[0, :, 0:1] + dp_ref[1, :, 0:1] + 1.0
    return lax.rsqrt(deg)


def _deg_spec():
    return pl.BlockSpec((NC, ROWBLK, 128), lambda i: (0, i, 0))


def _tc_scale_matmul(x_pad, W, deg_parts):
    n_pad, d = x_pad.shape

    def body(x_ref, w_ref, dp_ref, o_ref):
        dis = _dis_block(dp_ref)
        o_ref[...] = jnp.dot(x_ref[...], w_ref[...],
                             precision=lax.Precision.HIGHEST,
                             preferred_element_type=jnp.float32) * dis

    return pl.pallas_call(
        body,
        grid=(n_pad // ROWBLK,),
        in_specs=[
            pl.BlockSpec((ROWBLK, d), lambda i: (i, 0)),
            pl.BlockSpec((d, d), lambda i: (0, 0)),
            _deg_spec(),
        ],
        out_specs=pl.BlockSpec((ROWBLK, d), lambda i: (i, 0)),
        out_shape=jax.ShapeDtypeStruct((n_pad, d), jnp.float32),
    )(x_pad, W, deg_parts)


def _agg_block(p_ref, xw_ref):
    return p_ref[0] + p_ref[1] + xw_ref[...]


def _tc_mid(parts, xw_s, deg_parts, b, W):
    n_pad, d = xw_s.shape

    def body(p_ref, xw_ref, dp_ref, b_ref, w_ref, o_ref):
        dis = _dis_block(dp_ref)
        h = jnp.maximum(_agg_block(p_ref, xw_ref) * dis + b_ref[...], 0.0)
        o_ref[...] = jnp.dot(h, w_ref[...],
                             precision=lax.Precision.HIGHEST,
                             preferred_element_type=jnp.float32) * dis

    return pl.pallas_call(
        body,
        grid=(n_pad // ROWBLK,),
        in_specs=[
            pl.BlockSpec((NC, ROWBLK, d), lambda i: (0, i, 0)),
            pl.BlockSpec((ROWBLK, d), lambda i: (i, 0)),
            _deg_spec(),
            pl.BlockSpec((1, d), lambda i: (0, 0)),
            pl.BlockSpec((d, d), lambda i: (0, 0)),
        ],
        out_specs=pl.BlockSpec((ROWBLK, d), lambda i: (i, 0)),
        out_shape=jax.ShapeDtypeStruct((n_pad, d), jnp.float32),
    )(parts, xw_s, deg_parts, b, W)


def _tc_final(parts, xw_s, deg_parts, b, n):
    n_pad, d = xw_s.shape

    def body(p_ref, xw_ref, dp_ref, b_ref, o_ref):
        dis = _dis_block(dp_ref)
        o_ref[...] = _agg_block(p_ref, xw_ref) * dis + b_ref[...]

    return pl.pallas_call(
        body,
        grid=(n_pad // ROWBLK,),
        in_specs=[
            pl.BlockSpec((NC, ROWBLK, d), lambda i: (0, i, 0)),
            pl.BlockSpec((ROWBLK, d), lambda i: (i, 0)),
            _deg_spec(),
            pl.BlockSpec((1, d), lambda i: (0, 0)),
        ],
        out_specs=pl.BlockSpec((ROWBLK, d), lambda i: (i, 0)),
        out_shape=jax.ShapeDtypeStruct((n, d), jnp.float32),
    )(parts, xw_s, deg_parts, b)


def kernel(x, edge_index, W1, b1, W2, b2):
    n, d = x.shape
    e = edge_index.shape[1]

    n_pad = -(-(n + 1) // ROWBLK) * ROWBLK
    epw = -(-e // NW)
    nch = -(-epw // CHUNK)
    nch = -(-nch // (2 * NBUF)) * 2 * NBUF
    e_pad = NW * nch * CHUNK

    npad_e = e_pad - e
    ramp = jnp.arange(npad_e, dtype=edge_index.dtype)
    pad_src = n + ramp % (n_pad - n)
    pad_dst_agg = ramp % n_pad
    pad_dst_deg = n + ramp % (n_pad - n)
    src = jnp.concatenate([edge_index[0], pad_src]).reshape(NW, nch, CHUNK)
    dst = jnp.concatenate([edge_index[1], pad_dst_agg]).reshape(NW, nch, CHUNK)
    dstd = jnp.concatenate([edge_index[1], pad_dst_deg]).reshape(NW, nch, CHUNK)

    x_pad = jnp.zeros((n_pad, d), jnp.float32).at[:n].set(x)

    deg_parts = _sc_degree(n_pad, nch)(dstd)

    b1r = b1.reshape(1, d)
    b2r = b2.reshape(1, d)
    dh = d // 2
    srca = src * 2
    srcb = srca + 1

    agg = _sc_agg(n_pad, d, nch)

    xw1s = _tc_scale_matmul(x_pad, W1, deg_parts)
    parts1 = agg(xw1s.reshape(2 * n_pad, dh), srca, srcb, dst)
    xw2s = _tc_mid(parts1, xw1s, deg_parts, b1r, W2)
    parts2 = agg(xw2s.reshape(2 * n_pad, dh), srca, srcb, dst)
    return _tc_final(parts2, xw2s, deg_parts, b2r, n)


# speedup vs baseline: 31.7076x; 1.0359x over previous
import functools

import jax
import jax.numpy as jnp
from jax import lax
from jax.experimental import pallas as pl
from jax.experimental.pallas import tpu as pltpu
from jax.experimental.pallas import tpu_sc as plsc

NC = 2
NS = 16
NW = NC * NS
LANES = 16
CHUNK = 128
NBUF = 3
ROWBLK = 1280


def _mesh():
    return plsc.VectorSubcoreMesh(core_axis_name="c", subcore_axis_name="s")


@functools.lru_cache(maxsize=None)
def _sc_degree(n_pad: int, nch: int):
    rows_per_tile = n_pad // NS
    zcopies = rows_per_tile // CHUNK

    scratch = [
        pltpu.VMEM((nch, CHUNK), jnp.int32),
        pltpu.VMEM((CHUNK, LANES), jnp.float32),
        pltpu.VMEM((CHUNK, LANES), jnp.float32),
        pltpu.VMEM_SHARED((n_pad, LANES), jnp.float32),
        pltpu.SemaphoreType.DMA,
    ]

    @functools.partial(
        pl.kernel,
        out_type=jax.ShapeDtypeStruct((NC, n_pad, 128), jnp.float32),
        mesh=_mesh(),
        scratch_types=scratch,
        compiler_params=pltpu.CompilerParams(use_tc_tiling_on_sc=False),
    )
    def deg_k(dst_hbm, out_hbm, dst_v, ones_v, zero_v, acc, sem):
        c = lax.axis_index("c")
        s = lax.axis_index("s")
        wid = s * NC + c
        pltpu.sync_copy(dst_hbm.at[wid], dst_v)

        ones = jnp.ones((LANES,), jnp.float32)
        zeros = jnp.zeros((LANES,), jnp.float32)

        def fbody(i, carry):
            ones_v[i, :] = ones
            zero_v[i, :] = zeros
            return carry

        lax.fori_loop(0, CHUNK, fbody, 0)
        for k in range(zcopies):
            pltpu.sync_copy(
                zero_v, acc.at[pl.ds(s * rows_per_tile + k * CHUNK, CHUNK)])
        plsc.subcore_barrier()

        def fire(j, carry):
            pltpu.async_copy(ones_v, acc.at[dst_v.at[j]], sem, add=True)
            return carry

        lax.fori_loop(0, nch, fire, 0)

        def drain(j, carry):
            pltpu.make_async_copy(ones_v, acc.at[dst_v.at[j]], sem).wait()
            return carry

        lax.fori_loop(0, nch, drain, 0)
        plsc.subcore_barrier()
        pltpu.sync_copy(
            acc.at[pl.ds(s * rows_per_tile, rows_per_tile)],
            out_hbm.at[c].at[pl.ds(s * rows_per_tile, rows_per_tile),
                             pl.ds(0, LANES)])

    return deg_k


@functools.lru_cache(maxsize=None)
def _sc_agg(n_pad: int, d: int, nch: int):
    dh = d // 2
    rows_per_tile = n_pad // NS
    zcopies = rows_per_tile // CHUNK
    nb2 = 2 * NBUF

    scratch = [
        pltpu.VMEM((nch, CHUNK), jnp.int32),
        pltpu.VMEM((nch, CHUNK), jnp.int32),
        pltpu.VMEM((nch, CHUNK), jnp.int32),
        pltpu.VMEM((nb2, CHUNK, dh), jnp.float32),
        pltpu.VMEM((CHUNK, dh), jnp.float32),
        pltpu.VMEM_SHARED((n_pad, dh), jnp.float32),
    ] + [pltpu.SemaphoreType.DMA] * (2 * nb2)

    @functools.partial(
        pl.kernel,
        out_type=jax.ShapeDtypeStruct((NC, n_pad, d), jnp.float32),
        mesh=_mesh(),
        scratch_types=scratch,
        compiler_params=pltpu.CompilerParams(use_tc_tiling_on_sc=False),
    )
    def agg_k(xw_hbm, srca_hbm, srcb_hbm, dst_hbm, out_hbm, srca_v, srcb_v,
              dst_v, buf, zbuf, acc, *sems):
        gsems = sems[:nb2]
        ssems = sems[nb2:]
        c = lax.axis_index("c")
        s = lax.axis_index("s")
        wid = s * NC + c
        pltpu.sync_copy(srca_hbm.at[wid], srca_v)
        pltpu.sync_copy(srcb_hbm.at[wid], srcb_v)
        pltpu.sync_copy(dst_hbm.at[wid], dst_v)

        zeros = jnp.zeros((LANES,), jnp.float32)

        def zbody(i, carry):
            for k in range(dh // LANES):
                zbuf[i, pl.ds(k * LANES, LANES)] = zeros
            return carry

        lax.fori_loop(0, CHUNK, zbody, 0)

        for b in range(NBUF):
            pltpu.async_copy(xw_hbm.at[srca_v.at[b]], buf.at[b], gsems[b])

        for half in range(2):
            src_v = srca_v if half == 0 else srcb_v
            for k in range(zcopies):
                pltpu.sync_copy(
                    zbuf, acc.at[pl.ds(s * rows_per_tile + k * CHUNK, CHUNK)])
            plsc.subcore_barrier()

            def step(jo, carry):
                for u in range(nb2):
                    j = jo * nb2 + u
                    b = u
                    pltpu.make_async_copy(
                        xw_hbm.at[src_v.at[j]], buf.at[b], gsems[b]).wait()
                    pltpu.async_copy(
                        buf.at[b], acc.at[dst_v.at[j]], ssems[b], add=True)
                    jn = j + NBUF
                    bn = (u + NBUF) % nb2

                    @pl.when(jn < nch)
                    def _():
                        @pl.when(jn >= nb2)
                        def _():
                            pltpu.make_async_copy(
                                buf.at[bn], acc.at[dst_v.at[jn]],
                                ssems[bn]).wait()

                        pltpu.async_copy(
                            xw_hbm.at[src_v.at[jn]], buf.at[bn], gsems[bn])

                return carry

            lax.fori_loop(0, nch // nb2, step, 0)

            for u in range(nb2):
                j = nch - nb2 + u
                pltpu.make_async_copy(
                    buf.at[u], acc.at[dst_v.at[j]], ssems[u]).wait()

            if half == 0:
                for b in range(NBUF):
                    pltpu.async_copy(
                        xw_hbm.at[srcb_v.at[b]], buf.at[b], gsems[b])

            plsc.subcore_barrier()
            pltpu.sync_copy(
                acc.at[pl.ds(s * rows_per_tile, rows_per_tile)],
                out_hbm.at[c].at[pl.ds(s * rows_per_tile, rows_per_tile),
                                 pl.ds(half * dh, dh)])

    return agg_k


def _dis_block(dp_ref):
    deg = dp_ref[0, :, 0:1] + dp_ref[1, :, 0:1] + 1.0
    return lax.rsqrt(deg)


def _deg_spec():
    return pl.BlockSpec((NC, ROWBLK, 128), lambda i: (0, i, 0))


def _tc_scale_matmul(x_pad, W, deg_parts):
    n_pad, d = x_pad.shape

    def body(x_ref, w_ref, dp_ref, o_ref):
        dis = _dis_block(dp_ref)
        o_ref[...] = jnp.dot(x_ref[...], w_ref[...],
                             precision=lax.Precision.HIGHEST,
                             preferred_element_type=jnp.float32) * dis

    return pl.pallas_call(
        body,
        grid=(n_pad // ROWBLK,),
        in_specs=[
            pl.BlockSpec((ROWBLK, d), lambda i: (i, 0)),
            pl.BlockSpec((d, d), lambda i: (0, 0)),
            _deg_spec(),
        ],
        out_specs=pl.BlockSpec((ROWBLK, d), lambda i: (i, 0)),
        out_shape=jax.ShapeDtypeStruct((n_pad, d), jnp.float32),
    )(x_pad, W, deg_parts)


def _agg_block(p_ref, xw_ref):
    return p_ref[0] + p_ref[1] + xw_ref[...]


def _tc_mid(parts, xw_s, deg_parts, b, W):
    n_pad, d = xw_s.shape

    def body(p_ref, xw_ref, dp_ref, b_ref, w_ref, o_ref):
        dis = _dis_block(dp_ref)
        h = jnp.maximum(_agg_block(p_ref, xw_ref) * dis + b_ref[...], 0.0)
        o_ref[...] = jnp.dot(h, w_ref[...],
                             precision=lax.Precision.HIGHEST,
                             preferred_element_type=jnp.float32) * dis

    return pl.pallas_call(
        body,
        grid=(n_pad // ROWBLK,),
        in_specs=[
            pl.BlockSpec((NC, ROWBLK, d), lambda i: (0, i, 0)),
            pl.BlockSpec((ROWBLK, d), lambda i: (i, 0)),
            _deg_spec(),
            pl.BlockSpec((1, d), lambda i: (0, 0)),
            pl.BlockSpec((d, d), lambda i: (0, 0)),
        ],
        out_specs=pl.BlockSpec((ROWBLK, d), lambda i: (i, 0)),
        out_shape=jax.ShapeDtypeStruct((n_pad, d), jnp.float32),
    )(parts, xw_s, deg_parts, b, W)


def _tc_final(parts, xw_s, deg_parts, b, n):
    n_pad, d = xw_s.shape

    def body(p_ref, xw_ref, dp_ref, b_ref, o_ref):
        dis = _dis_block(dp_ref)
        o_ref[...] = _agg_block(p_ref, xw_ref) * dis + b_ref[...]

    return pl.pallas_call(
        body,
        grid=(n_pad // ROWBLK,),
        in_specs=[
            pl.BlockSpec((NC, ROWBLK, d), lambda i: (0, i, 0)),
            pl.BlockSpec((ROWBLK, d), lambda i: (i, 0)),
            _deg_spec(),
            pl.BlockSpec((1, d), lambda i: (0, 0)),
        ],
        out_specs=pl.BlockSpec((ROWBLK, d), lambda i: (i, 0)),
        out_shape=jax.ShapeDtypeStruct((n, d), jnp.float32),
    )(parts, xw_s, deg_parts, b)


def kernel(x, edge_index, W1, b1, W2, b2):
    n, d = x.shape
    e = edge_index.shape[1]

    n_pad = -(-(n + 1) // ROWBLK) * ROWBLK
    epw = -(-e // NW)
    nch = -(-epw // CHUNK)
    nch = -(-nch // (2 * NBUF)) * 2 * NBUF
    e_pad = NW * nch * CHUNK

    npad_e = e_pad - e
    ramp = jnp.arange(npad_e, dtype=edge_index.dtype)
    pad_src = n + ramp % (n_pad - n)
    pad_dst_agg = ramp % n_pad
    pad_dst_deg = n + ramp % (n_pad - n)
    src = jnp.concatenate([edge_index[0], pad_src]).reshape(NW, nch, CHUNK)
    dst = jnp.concatenate([edge_index[1], pad_dst_agg]).reshape(NW, nch, CHUNK)
    dstd = jnp.concatenate([edge_index[1], pad_dst_deg]).reshape(NW, nch, CHUNK)

    x_pad = jnp.zeros((n_pad, d), jnp.float32).at[:n].set(x)

    deg_parts = _sc_degree(n_pad, nch)(dstd)

    b1r = b1.reshape(1, d)
    b2r = b2.reshape(1, d)
    dh = d // 2
    srca = src * 2
    srcb = srca + 1

    agg = _sc_agg(n_pad, d, nch)

    xw1s = _tc_scale_matmul(x_pad, W1, deg_parts)
    parts1 = agg(xw1s.reshape(2 * n_pad, dh), srca, srcb, dst)
    xw2s = _tc_mid(parts1, xw1s, deg_parts, b1r, W2)
    parts2 = agg(xw2s.reshape(2 * n_pad, dh), srca, srcb, dst)
    return _tc_final(parts2, xw2s, deg_parts, b2r, n)


# speedup vs baseline: 33.0326x; 1.0418x over previous
import functools

import jax
import jax.numpy as jnp
from jax import lax
from jax.experimental import pallas as pl
from jax.experimental.pallas import tpu as pltpu
from jax.experimental.pallas import tpu_sc as plsc

NC = 2
NS = 16
NW = NC * NS
LANES = 16
CHUNK = 128
NBUF = 3
ROWBLK = 1280


def _mesh():
    return plsc.VectorSubcoreMesh(core_axis_name="c", subcore_axis_name="s")


@functools.lru_cache(maxsize=None)
def _sc_degree(n_pad: int, nch: int):
    rows_per_tile = n_pad // NS
    zcopies = rows_per_tile // CHUNK

    scratch = [
        pltpu.VMEM((nch, CHUNK), jnp.int32),
        pltpu.VMEM((CHUNK, LANES), jnp.float32),
        pltpu.VMEM((CHUNK, LANES), jnp.float32),
        pltpu.VMEM_SHARED((n_pad, LANES), jnp.float32),
        pltpu.SemaphoreType.DMA,
    ]

    @functools.partial(
        pl.kernel,
        out_type=jax.ShapeDtypeStruct((NC, n_pad, 128), jnp.float32),
        mesh=_mesh(),
        scratch_types=scratch,
        compiler_params=pltpu.CompilerParams(use_tc_tiling_on_sc=False),
    )
    def deg_k(dst_hbm, out_hbm, dst_v, ones_v, zero_v, acc, sem):
        c = lax.axis_index("c")
        s = lax.axis_index("s")
        wid = s * NC + c
        pltpu.sync_copy(dst_hbm.at[wid], dst_v)

        ones = jnp.ones((LANES,), jnp.float32)
        zeros = jnp.zeros((LANES,), jnp.float32)

        def fbody(i, carry):
            ones_v[i, :] = ones
            zero_v[i, :] = zeros
            return carry

        lax.fori_loop(0, CHUNK, fbody, 0)
        for k in range(zcopies):
            pltpu.sync_copy(
                zero_v, acc.at[pl.ds(s * rows_per_tile + k * CHUNK, CHUNK)])
        plsc.subcore_barrier()

        def fire(j, carry):
            pltpu.async_copy(ones_v, acc.at[dst_v.at[j]], sem, add=True)
            return carry

        lax.fori_loop(0, nch, fire, 0)

        def drain(j, carry):
            pltpu.make_async_copy(ones_v, acc.at[dst_v.at[j]], sem).wait()
            return carry

        lax.fori_loop(0, nch, drain, 0)
        plsc.subcore_barrier()
        pltpu.sync_copy(
            acc.at[pl.ds(s * rows_per_tile, rows_per_tile)],
            out_hbm.at[c].at[pl.ds(s * rows_per_tile, rows_per_tile),
                             pl.ds(0, LANES)])

    return deg_k


@functools.lru_cache(maxsize=None)
def _sc_agg(n_pad: int, d: int, nch: int):
    dh = d // 2
    rows_per_tile = n_pad // NS
    zcopies = rows_per_tile // CHUNK
    nb2 = 2 * NBUF

    scratch = [
        pltpu.VMEM((nch, CHUNK), jnp.int32),
        pltpu.VMEM((nch, CHUNK), jnp.int32),
        pltpu.VMEM((nch, CHUNK), jnp.int32),
        pltpu.VMEM((nb2, CHUNK, dh), jnp.float32),
        pltpu.VMEM((CHUNK, dh), jnp.float32),
        pltpu.VMEM_SHARED((n_pad, dh), jnp.float32),
    ] + [pltpu.SemaphoreType.DMA] * (2 * nb2)

    @functools.partial(
        pl.kernel,
        out_type=jax.ShapeDtypeStruct((NC, n_pad, d), jnp.float32),
        mesh=_mesh(),
        scratch_types=scratch,
        compiler_params=pltpu.CompilerParams(use_tc_tiling_on_sc=False),
    )
    def agg_k(xw_hbm, srca_hbm, srcb_hbm, dst_hbm, out_hbm, srca_v, srcb_v,
              dst_v, buf, zbuf, acc, *sems):
        gsems = sems[:nb2]
        ssems = sems[nb2:]
        c = lax.axis_index("c")
        s = lax.axis_index("s")
        wid = s * NC + c
        pltpu.sync_copy(srca_hbm.at[wid], srca_v)
        pltpu.sync_copy(srcb_hbm.at[wid], srcb_v)
        pltpu.sync_copy(dst_hbm.at[wid], dst_v)

        zeros = jnp.zeros((LANES,), jnp.float32)

        def zbody(i, carry):
            for k in range(dh // LANES):
                zbuf[i, pl.ds(k * LANES, LANES)] = zeros
            return carry

        lax.fori_loop(0, CHUNK, zbody, 0)

        for b in range(NBUF):
            pltpu.async_copy(xw_hbm.at[srca_v.at[b]], buf.at[b], gsems[b])

        for half in range(2):
            src_v = srca_v if half == 0 else srcb_v
            for k in range(zcopies):
                pltpu.sync_copy(
                    zbuf, acc.at[pl.ds(s * rows_per_tile + k * CHUNK, CHUNK)])
            plsc.subcore_barrier()

            def step(jo, carry):
                for u in range(nb2):
                    j = jo * nb2 + u
                    b = u

                    @pl.when(j < nch)
                    def _():
                        pltpu.make_async_copy(
                            xw_hbm.at[src_v.at[j]], buf.at[b],
                            gsems[b]).wait()
                        pltpu.async_copy(
                            buf.at[b], acc.at[dst_v.at[j]], ssems[b],
                            add=True)
                        jn = j + NBUF
                        bn = (u + NBUF) % nb2

                        @pl.when(jn < nch)
                        def _():
                            @pl.when(jn >= nb2)
                            def _():
                                pltpu.make_async_copy(
                                    buf.at[bn], acc.at[dst_v.at[jn]],
                                    ssems[bn]).wait()

                            pltpu.async_copy(
                                xw_hbm.at[src_v.at[jn]], buf.at[bn],
                                gsems[bn])

                return carry

            lax.fori_loop(0, -(-nch // nb2), step, 0)

            for u in range(nb2):
                j = nch - nb2 + u
                b = j % nb2
                pltpu.make_async_copy(
                    buf.at[b], acc.at[dst_v.at[j]], ssems[b]).wait()

            if half == 0:
                for b in range(NBUF):
                    pltpu.async_copy(
                        xw_hbm.at[srcb_v.at[b]], buf.at[b], gsems[b])

            plsc.subcore_barrier()
            pltpu.sync_copy(
                acc.at[pl.ds(s * rows_per_tile, rows_per_tile)],
                out_hbm.at[c].at[pl.ds(s * rows_per_tile, rows_per_tile),
                                 pl.ds(half * dh, dh)])

    return agg_k


def _dis_block(dp_ref):
    deg = dp_ref[0, :, 0:1] + dp_ref[1, :, 0:1] + 1.0
    return lax.rsqrt(deg)


def _deg_spec():
    return pl.BlockSpec((NC, ROWBLK, 128), lambda i: (0, i, 0))


def _tc_scale_matmul(x_pad, W, deg_parts):
    n_pad, d = x_pad.shape

    def body(x_ref, w_ref, dp_ref, o_ref):
        dis = _dis_block(dp_ref)
        o_ref[...] = jnp.dot(x_ref[...], w_ref[...],
                             precision=lax.Precision.HIGHEST,
                             preferred_element_type=jnp.float32) * dis

    return pl.pallas_call(
        body,
        grid=(n_pad // ROWBLK,),
        in_specs=[
            pl.BlockSpec((ROWBLK, d), lambda i: (i, 0)),
            pl.BlockSpec((d, d), lambda i: (0, 0)),
            _deg_spec(),
        ],
        out_specs=pl.BlockSpec((ROWBLK, d), lambda i: (i, 0)),
        out_shape=jax.ShapeDtypeStruct((n_pad, d), jnp.float32),
    )(x_pad, W, deg_parts)


def _agg_block(p_ref, xw_ref):
    return p_ref[0] + p_ref[1] + xw_ref[...]


def _tc_mid(parts, xw_s, deg_parts, b, W):
    n_pad, d = xw_s.shape

    def body(p_ref, xw_ref, dp_ref, b_ref, w_ref, o_ref):
        dis = _dis_block(dp_ref)
        h = jnp.maximum(_agg_block(p_ref, xw_ref) * dis + b_ref[...], 0.0)
        o_ref[...] = jnp.dot(h, w_ref[...],
                             precision=lax.Precision.HIGHEST,
                             preferred_element_type=jnp.float32) * dis

    return pl.pallas_call(
        body,
        grid=(n_pad // ROWBLK,),
        in_specs=[
            pl.BlockSpec((NC, ROWBLK, d), lambda i: (0, i, 0)),
            pl.BlockSpec((ROWBLK, d), lambda i: (i, 0)),
            _deg_spec(),
            pl.BlockSpec((1, d), lambda i: (0, 0)),
            pl.BlockSpec((d, d), lambda i: (0, 0)),
        ],
        out_specs=pl.BlockSpec((ROWBLK, d), lambda i: (i, 0)),
        out_shape=jax.ShapeDtypeStruct((n_pad, d), jnp.float32),
    )(parts, xw_s, deg_parts, b, W)


def _tc_final(parts, xw_s, deg_parts, b, n):
    n_pad, d = xw_s.shape

    def body(p_ref, xw_ref, dp_ref, b_ref, o_ref):
        dis = _dis_block(dp_ref)
        o_ref[...] = _agg_block(p_ref, xw_ref) * dis + b_ref[...]

    return pl.pallas_call(
        body,
        grid=(n_pad // ROWBLK,),
        in_specs=[
            pl.BlockSpec((NC, ROWBLK, d), lambda i: (0, i, 0)),
            pl.BlockSpec((ROWBLK, d), lambda i: (i, 0)),
            _deg_spec(),
            pl.BlockSpec((1, d), lambda i: (0, 0)),
        ],
        out_specs=pl.BlockSpec((ROWBLK, d), lambda i: (i, 0)),
        out_shape=jax.ShapeDtypeStruct((n, d), jnp.float32),
    )(parts, xw_s, deg_parts, b)


def kernel(x, edge_index, W1, b1, W2, b2):
    n, d = x.shape
    e = edge_index.shape[1]

    n_pad = -(-(n + 1) // ROWBLK) * ROWBLK
    epw = -(-e // NW)
    nch = max(-(-epw // CHUNK), 2 * NBUF)
    e_pad = NW * nch * CHUNK

    npad_e = e_pad - e
    ramp = jnp.arange(npad_e, dtype=edge_index.dtype)
    pad_src = n + ramp % (n_pad - n)
    pad_dst_agg = ramp % n_pad
    pad_dst_deg = n + ramp % (n_pad - n)
    src = jnp.concatenate([edge_index[0], pad_src]).reshape(NW, nch, CHUNK)
    dst = jnp.concatenate([edge_index[1], pad_dst_agg]).reshape(NW, nch, CHUNK)
    dstd = jnp.concatenate([edge_index[1], pad_dst_deg]).reshape(NW, nch, CHUNK)

    x_pad = jnp.zeros((n_pad, d), jnp.float32).at[:n].set(x)

    deg_parts = _sc_degree(n_pad, nch)(dstd)

    b1r = b1.reshape(1, d)
    b2r = b2.reshape(1, d)
    dh = d // 2
    srca = src * 2
    srcb = srca + 1

    agg = _sc_agg(n_pad, d, nch)

    xw1s = _tc_scale_matmul(x_pad, W1, deg_parts)
    parts1 = agg(xw1s.reshape(2 * n_pad, dh), srca, srcb, dst)
    xw2s = _tc_mid(parts1, xw1s, deg_parts, b1r, W2)
    parts2 = agg(xw2s.reshape(2 * n_pad, dh), srca, srcb, dst)
    return _tc_final(parts2, xw2s, deg_parts, b2r, n)
